# baseline (speedup 1.0000x reference)
import functools

import jax
import jax.numpy as jnp
from jax import lax
from jax.experimental import pallas as pl
from jax.experimental.pallas import tpu as pltpu
from jax.experimental.pallas import tpu_sc as plsc

EMBED_DIM = 64
NUM_CORES = 2
NUM_SUBCORES = 16
NUM_WORKERS = NUM_CORES * NUM_SUBCORES
CHUNK = 128
NBUF = 5


@jax.jit
def _embedding_lookup(idx_t, table):
    seq, batch = idx_t.shape
    total = seq * batch
    blocks_per_row = batch // CHUNK
    per_worker = total // NUM_WORKERS
    n_chunks = per_worker // CHUNK
    n_groups = n_chunks // NBUF

    mesh = plsc.VectorSubcoreMesh(
        core_axis_name="c",
        subcore_axis_name="s",
        num_cores=NUM_CORES,
        num_subcores=NUM_SUBCORES,
    )

    @functools.partial(
        pl.kernel,
        mesh=mesh,
        compiler_params=pltpu.CompilerParams(use_tc_tiling_on_sc=False),
        out_type=jax.ShapeDtypeStruct((total, EMBED_DIM), jnp.float32),
        scratch_types=(
            [pltpu.VMEM((per_worker,), jnp.int32)]
            + [pltpu.VMEM((CHUNK, EMBED_DIM), jnp.float32) for _ in range(NBUF)]
            + [pltpu.SemaphoreType.DMA for _ in range(2 * NBUF)]
            + [pltpu.SemaphoreType.DMA]
        ),
    )
    def emb(idx_hbm, table_hbm, out_hbm, idx_v, *rest):
        bufs = rest[:NBUF]
        sem_g = rest[NBUF : 2 * NBUF]
        sem_w = rest[2 * NBUF : 3 * NBUF]
        sem_i = rest[3 * NBUF]

        wid = lax.axis_index("s") * NUM_CORES + lax.axis_index("c")
        base = wid * per_worker
        t0 = wid * n_chunks

        descs = []
        for k in range(n_chunks):
            t = t0 + k
            descs.append(
                pltpu.make_async_copy(
                    idx_hbm.at[
                        t // blocks_per_row,
                        pl.ds((t % blocks_per_row) * CHUNK, CHUNK),
                    ],
                    idx_v.at[pl.ds(k * CHUNK, CHUNK)],
                    sem_i,
                )
            )
        for d in descs:
            d.start()
        for d in descs:
            d.wait()

        def gather(k, b):
            return pltpu.make_async_copy(
                table_hbm.at[idx_v.at[pl.ds(k * CHUNK, CHUNK)]], bufs[b], sem_g[b]
            )

        def writeback(k, b):
            return pltpu.make_async_copy(
                bufs[b], out_hbm.at[pl.ds(base + k * CHUNK, CHUNK)], sem_w[b]
            )

        for b in range(NBUF):
            gather(b, b).start()

        def group(g, carry):
            j0 = g * NBUF
            for b in range(NBUF):
                gather(j0 + b, b).wait()
                writeback(j0 + b, b).start()
            for b in range(NBUF):
                writeback(j0 + b, b).wait()
                gather(j0 + NBUF + b, b).start()
            return carry

        lax.fori_loop(0, n_groups - 1, group, 0)

        j0 = (n_groups - 1) * NBUF
        for b in range(NBUF):
            gather(j0 + b, b).wait()
            writeback(j0 + b, b).start()
        for b in range(NBUF):
            writeback(j0 + b, b).wait()

    return emb(idx_t, table)


def kernel(input_sentence, W):
    batch, seq = input_sentence.shape
    idx_t = input_sentence.T.astype(jnp.int32)
    out = _embedding_lookup(idx_t, W)
    return out.reshape(seq, batch, EMBED_DIM).transpose(1, 0, 2)
